# baseline (speedup 1.0000x reference)
import functools

import jax
import jax.numpy as jnp
from jax import lax
from jax.experimental import pallas as pl
from jax.experimental.pallas import tpu as pltpu
from jax.experimental.pallas import tpu_sc as plsc

_NC = 2
_NS = 16
_NW = _NC * _NS
_L = 16

_D = 96
_ROWS = 64
_REP = 4
_SEC = _ROWS * _REP + _L
_CHK = 384


def _build_table_t(W_aromatic, W_conjugated, W_stereo):
    r = jnp.arange(_ROWS - 1)
    tab = jnp.concatenate(
        [W_aromatic[r // 21], W_conjugated[(r // 7) % 3], W_stereo[r % 7]],
        axis=1,
    )
    tab = jnp.concatenate([tab, jnp.zeros((1, _D), jnp.float32)], axis=0)
    rep = jnp.repeat(tab.T, _REP, axis=1)
    rep = jnp.concatenate(
        [rep, jnp.zeros((_D, _L), jnp.float32)], axis=1)
    return rep.reshape(-1)


@functools.partial(jax.jit, static_argnames=("E",))
def _sc_lookup(idx_a, idx_c, idx_s, mask_i32, table_t, *, E):
    n_total = -(-E // _CHK)
    last = n_total - 1
    n_per_w = -(-n_total // _NW)
    n_per_w += n_per_w % 2
    mesh = plsc.VectorSubcoreMesh(core_axis_name="c", subcore_axis_name="s")

    @functools.partial(
        pl.kernel,
        out_type=jax.ShapeDtypeStruct((_D, E), jnp.float32),
        mesh=mesh,
        scratch_types=[
            pltpu.VMEM((_D * _SEC,), jnp.float32),
            [pltpu.VMEM((4 * _CHK,), jnp.int32)] * 2,
            [pltpu.VMEM((_D, _CHK), jnp.float32)] * 2,
            [pltpu.SemaphoreType.DMA] * 2,
            [pltpu.SemaphoreType.DMA] * 2,
            pltpu.SemaphoreType.DMA,
        ],
        compiler_params=pltpu.CompilerParams(
            needs_layout_passes=False, disable_bounds_checks=True),
    )
    def body(a_hbm, c_hbm, s_hbm, m_hbm, tab_hbm, out_hbm,
             tab_v, idx_v, out_v, isem, osem, tsem):
        wid = lax.axis_index("s") * _NC + lax.axis_index("c")

        tab_cp = pltpu.async_copy(tab_hbm, tab_v, tsem)

        def chunk_base(i):
            cid = jnp.minimum(wid + i * _NW, last)
            return jnp.minimum(cid * _CHK, E - _CHK)

        def issue_idx(i, b):
            base = chunk_base(i)
            for q, src in enumerate((a_hbm, c_hbm, s_hbm, m_hbm)):
                pltpu.async_copy(
                    src.at[pl.ds(base, _CHK)],
                    idx_v[b].at[pl.ds(q * _CHK, _CHK)],
                    isem[b],
                )

        issue_idx(0, 0)
        tab_cp.wait()

        @pl.loop(0, n_per_w, step=2)
        def chunk_pair(i):
            for b in (0, 1):
                j = i + b
                issue_idx(j + 1, 1 - b)
                for _ in range(4):
                    pltpu.make_async_copy(
                        a_hbm.at[pl.ds(0, _CHK)],
                        idx_v[b].at[pl.ds(0, _CHK)],
                        isem[b],
                    ).wait()

                @pl.when(j >= 2)
                def _():
                    pltpu.make_async_copy(
                        out_v[b],
                        out_hbm.at[:, pl.ds(0, _CHK)],
                        osem[b],
                    ).wait()

                lanes = lax.iota(jnp.int32, _L)

                @plsc.parallel_loop(0, _CHK // _L, unroll=2)
                def group_loop(g):
                    sl = lambda q: pl.ds(q * _CHK + g * _L, _L)
                    comb = (idx_v[b][sl(0)] * 21 + idx_v[b][sl(1)] * 7
                            + idx_v[b][sl(2)]) * _REP + (lanes & (_REP - 1))
                    comb = jnp.where(idx_v[b][sl(3)] != 0, comb,
                                     _ROWS * _REP + lanes)
                    for c in range(_D):
                        out_v[b][c, pl.ds(g * _L, _L)] = plsc.load_gather(
                            tab_v.at[pl.ds(c * _SEC, _SEC)], [comb])

                base = chunk_base(j)
                pltpu.async_copy(
                    out_v[b],
                    out_hbm.at[:, pl.ds(base, _CHK)],
                    osem[b],
                )

        for b in (0, 1):
            pltpu.make_async_copy(
                out_v[b], out_hbm.at[:, pl.ds(0, _CHK)], osem[b],
            ).wait()
        for _ in range(4):
            pltpu.make_async_copy(
                a_hbm.at[pl.ds(0, _CHK)],
                idx_v[0].at[pl.ds(0, _CHK)],
                isem[0],
            ).wait()

    return body(idx_a, idx_c, idx_s, mask_i32, table_t)


def kernel(bond_mask, prop_bond_aromatic, prop_bond_conjugated,
           prop_bond_stereo, W_aromatic, W_conjugated, W_stereo):
    E = bond_mask.shape[0]
    table_t = _build_table_t(W_aromatic, W_conjugated, W_stereo)
    out_t = _sc_lookup(
        prop_bond_aromatic.astype(jnp.int32),
        prop_bond_conjugated.astype(jnp.int32),
        prop_bond_stereo.astype(jnp.int32),
        bond_mask.astype(jnp.int32),
        table_t,
        E=E,
    )
    return out_t.T


# speedup vs baseline: 1.3445x; 1.3445x over previous
import functools

import jax
import jax.numpy as jnp
from jax import lax
from jax.experimental import pallas as pl
from jax.experimental.pallas import tpu as pltpu
from jax.experimental.pallas import tpu_sc as plsc

_NC = 2
_NS = 16
_NW = _NC * _NS
_L = 16

_D = 96
_ROWS = 64
_REP = 4
_SEC = _ROWS * _REP + _L
_CHK = 512


def _build_table_t(W_aromatic, W_conjugated, W_stereo):
    r = jnp.arange(_ROWS - 1)
    tab = jnp.concatenate(
        [W_aromatic[r // 21], W_conjugated[(r // 7) % 3], W_stereo[r % 7]],
        axis=1,
    )
    tab = jnp.concatenate([tab, jnp.zeros((1, _D), jnp.float32)], axis=0)
    rep = jnp.repeat(tab.T, _REP, axis=1)
    rep = jnp.concatenate(
        [rep, jnp.zeros((_D, _L), jnp.float32)], axis=1)
    return rep.reshape(-1)


@functools.partial(jax.jit, static_argnames=("E",))
def _sc_lookup(idx_a, idx_c, idx_s, mask_i32, table_t, *, E):
    n_total = -(-E // _CHK)
    last = n_total - 1
    n_per_w = -(-n_total // _NW)
    n_per_w += n_per_w % 2
    mesh = plsc.VectorSubcoreMesh(core_axis_name="c", subcore_axis_name="s")

    @functools.partial(
        pl.kernel,
        out_type=jax.ShapeDtypeStruct((_D, E), jnp.float32),
        mesh=mesh,
        scratch_types=[
            pltpu.VMEM((_D * _SEC,), jnp.float32),
            [pltpu.VMEM((4 * _CHK,), jnp.int32)] * 2,
            [pltpu.VMEM((_D, _CHK), jnp.float32)] * 2,
            [pltpu.SemaphoreType.DMA] * 2,
            [pltpu.SemaphoreType.DMA] * 2,
            pltpu.SemaphoreType.DMA,
        ],
        compiler_params=pltpu.CompilerParams(
            needs_layout_passes=False, disable_bounds_checks=True),
    )
    def body(a_hbm, c_hbm, s_hbm, m_hbm, tab_hbm, out_hbm,
             tab_v, idx_v, out_v, isem, osem, tsem):
        wid = lax.axis_index("s") * _NC + lax.axis_index("c")

        tab_cp = pltpu.async_copy(tab_hbm, tab_v, tsem)

        def chunk_base(i):
            cid = jnp.minimum(wid + i * _NW, last)
            return jnp.minimum(cid * _CHK, E - _CHK)

        def issue_idx(i, b):
            base = chunk_base(i)
            for q, src in enumerate((a_hbm, c_hbm, s_hbm, m_hbm)):
                pltpu.async_copy(
                    src.at[pl.ds(base, _CHK)],
                    idx_v[b].at[pl.ds(q * _CHK, _CHK)],
                    isem[b],
                )

        issue_idx(0, 0)
        tab_cp.wait()

        @pl.loop(0, n_per_w, step=2)
        def chunk_pair(i):
            for b in (0, 1):
                j = i + b
                issue_idx(j + 1, 1 - b)
                for _ in range(4):
                    pltpu.make_async_copy(
                        a_hbm.at[pl.ds(0, _CHK)],
                        idx_v[b].at[pl.ds(0, _CHK)],
                        isem[b],
                    ).wait()

                @pl.when(j >= 2)
                def _():
                    pltpu.make_async_copy(
                        out_v[b],
                        out_hbm.at[:, pl.ds(0, _CHK)],
                        osem[b],
                    ).wait()

                lanes = lax.iota(jnp.int32, _L)

                @plsc.parallel_loop(0, _CHK // _L)
                def group_loop(g):
                    sl = lambda q: pl.ds(q * _CHK + g * _L, _L)
                    comb = (idx_v[b][sl(0)] * 21 + idx_v[b][sl(1)] * 7
                            + idx_v[b][sl(2)]) * _REP + (lanes & (_REP - 1))
                    comb = jnp.where(idx_v[b][sl(3)] != 0, comb,
                                     _ROWS * _REP + lanes)
                    for c in range(_D):
                        out_v[b][c, pl.ds(g * _L, _L)] = plsc.load_gather(
                            tab_v.at[pl.ds(c * _SEC, _SEC)], [comb])

                base = chunk_base(j)
                pltpu.async_copy(
                    out_v[b],
                    out_hbm.at[:, pl.ds(base, _CHK)],
                    osem[b],
                )

        for b in (0, 1):
            pltpu.make_async_copy(
                out_v[b], out_hbm.at[:, pl.ds(0, _CHK)], osem[b],
            ).wait()
        for _ in range(4):
            pltpu.make_async_copy(
                a_hbm.at[pl.ds(0, _CHK)],
                idx_v[0].at[pl.ds(0, _CHK)],
                isem[0],
            ).wait()

    return body(idx_a, idx_c, idx_s, mask_i32, table_t)


def kernel(bond_mask, prop_bond_aromatic, prop_bond_conjugated,
           prop_bond_stereo, W_aromatic, W_conjugated, W_stereo):
    E = bond_mask.shape[0]
    table_t = _build_table_t(W_aromatic, W_conjugated, W_stereo)
    out_t = _sc_lookup(
        prop_bond_aromatic.astype(jnp.int32),
        prop_bond_conjugated.astype(jnp.int32),
        prop_bond_stereo.astype(jnp.int32),
        bond_mask.astype(jnp.int32),
        table_t,
        E=E,
    )
    return out_t.T
